# baseline (speedup 1.0000x reference)
import functools

import jax
import jax.numpy as jnp
import numpy as np
from jax.experimental import pallas as pl
from jax.experimental.pallas import tpu as pltpu

_SCALE = 2
_ROWS_PER_BLK = 8


def _route_indices(weight, out_channels, total):
    t2 = np.float32(np.float64(total) * np.float64(total))
    p = weight[:out_channels].astype(jnp.float32) * t2
    t = jnp.trunc(p)
    oc = np.float32(out_channels)
    r = jnp.fmod(t, oc)
    r = jnp.where(r < 0, r + oc, r).astype(jnp.int32)
    return r[:, 0], r[:, 1]


def _body(w_ref, s_ref, xs_ref, x_ref, o_ref, *, rb, tpr, cc):
    b = pl.program_id(0)
    j = pl.program_id(1)
    for rr in range(rb):
        row = j * rb + rr

        @pl.when(w_ref[row] == 0)
        def _copy():
            o_ref[0, rr] = x_ref[0, rr]

        @pl.when(w_ref[row] != 0)
        def _fill():
            v = xs_ref[b, s_ref[row]]
            o_ref[0, rr] = jnp.full((tpr, cc), v, jnp.float32)


def kernel(x, weight):
    B, T, C = x.shape
    oc = C // _SCALE
    ots = T * _SCALE
    total = C * T - 1
    tpr = T // oc

    idx1, idx2 = _route_indices(weight, oc, total)
    ii = jnp.arange(oc, dtype=jnp.int32)
    last_i = jnp.full((oc,), -1, jnp.int32).at[idx1].max(ii)
    written = (last_i >= 0).astype(jnp.int32)
    srcs = idx2[jnp.clip(last_i, 0, oc - 1)]

    xs = x[:, 0, :oc]
    x4 = x.reshape(B, oc, tpr, C)

    rb = _ROWS_PER_BLK
    grid = (B, oc // rb)
    blk = (1, rb, tpr, C)

    out4 = pl.pallas_call(
        functools.partial(_body, rb=rb, tpr=tpr, cc=C),
        grid=grid,
        in_specs=[
            pl.BlockSpec(memory_space=pltpu.SMEM),
            pl.BlockSpec(memory_space=pltpu.SMEM),
            pl.BlockSpec(memory_space=pltpu.SMEM),
            pl.BlockSpec(blk, lambda b, j: (b, j, 0, 0)),
        ],
        out_specs=pl.BlockSpec(blk, lambda b, j: (b, j, 0, 0)),
        out_shape=jax.ShapeDtypeStruct((B, oc, tpr, C), jnp.float32),
        compiler_params=pltpu.CompilerParams(
            dimension_semantics=("parallel", "arbitrary"),
        ),
    )(written, srcs, xs, x4)
    return out4.reshape(B, oc, ots)


# speedup vs baseline: 1.4845x; 1.4845x over previous
import functools

import jax
import jax.numpy as jnp
import numpy as np
from jax.experimental import pallas as pl
from jax.experimental.pallas import tpu as pltpu

_SCALE = 2
_ROWS_PER_BLK = 32


def _route_indices(weight, out_channels, total):
    t2 = np.float32(np.float64(total) * np.float64(total))
    p = weight[:out_channels].astype(jnp.float32) * t2
    t = jnp.trunc(p)
    oc = np.float32(out_channels)
    r = jnp.fmod(t, oc)
    r = jnp.where(r < 0, r + oc, r).astype(jnp.int32)
    return r[:, 0], r[:, 1]


def _body(w_ref, s_ref, xs_ref, x_ref, o_ref, *, rb, tpr, cc):
    b = pl.program_id(0)
    j = pl.program_id(1)
    for rr in range(rb):
        row = j * rb + rr

        @pl.when(w_ref[row] == 0)
        def _copy():
            o_ref[0, rr] = x_ref[0, rr]

        @pl.when(w_ref[row] != 0)
        def _fill():
            v = xs_ref[b, s_ref[row]]
            o_ref[0, rr] = jnp.full((tpr, cc), v, jnp.float32)


def kernel(x, weight):
    B, T, C = x.shape
    oc = C // _SCALE
    ots = T * _SCALE
    total = C * T - 1
    tpr = T // oc

    idx1, idx2 = _route_indices(weight, oc, total)
    ii = jnp.arange(oc, dtype=jnp.int32)
    last_i = jnp.full((oc,), -1, jnp.int32).at[idx1].max(ii)
    written = (last_i >= 0).astype(jnp.int32)
    srcs = idx2[jnp.clip(last_i, 0, oc - 1)]

    xs = x[:, 0, :oc]
    x4 = x.reshape(B, oc, tpr, C)

    rb = _ROWS_PER_BLK
    grid = (B, oc // rb)
    blk = (1, rb, tpr, C)

    out4 = pl.pallas_call(
        functools.partial(_body, rb=rb, tpr=tpr, cc=C),
        grid=grid,
        in_specs=[
            pl.BlockSpec(memory_space=pltpu.SMEM),
            pl.BlockSpec(memory_space=pltpu.SMEM),
            pl.BlockSpec(memory_space=pltpu.SMEM),
            pl.BlockSpec(blk, lambda b, j: (b, j, 0, 0)),
        ],
        out_specs=pl.BlockSpec(blk, lambda b, j: (b, j, 0, 0)),
        out_shape=jax.ShapeDtypeStruct((B, oc, tpr, C), jnp.float32),
        compiler_params=pltpu.CompilerParams(
            dimension_semantics=("parallel", "arbitrary"),
        ),
    )(written, srcs, xs, x4)
    return out4.reshape(B, oc, ots)


# speedup vs baseline: 1.6663x; 1.1224x over previous
import functools

import jax
import jax.numpy as jnp
import numpy as np
from jax.experimental import pallas as pl
from jax.experimental.pallas import tpu as pltpu

_SCALE = 2
_ROWS_PER_BLK = 64


def _route_indices(weight, out_channels, total):
    t2 = np.float32(np.float64(total) * np.float64(total))
    p = weight[:out_channels].astype(jnp.float32) * t2
    t = jnp.trunc(p)
    oc = np.float32(out_channels)
    r = jnp.fmod(t, oc)
    r = jnp.where(r < 0, r + oc, r).astype(jnp.int32)
    return r[:, 0], r[:, 1]


def _body(w_ref, s_ref, xs_ref, x_ref, o_ref, *, rb, tpr, cc):
    b = pl.program_id(0)
    j = pl.program_id(1)
    for rr in range(rb):
        row = j * rb + rr

        @pl.when(w_ref[row] == 0)
        def _copy():
            o_ref[0, rr] = x_ref[0, rr]

        @pl.when(w_ref[row] != 0)
        def _fill():
            v = xs_ref[b, s_ref[row]]
            o_ref[0, rr] = jnp.full((tpr, cc), v, jnp.float32)


def kernel(x, weight):
    B, T, C = x.shape
    oc = C // _SCALE
    ots = T * _SCALE
    total = C * T - 1
    tpr = T // oc

    idx1, idx2 = _route_indices(weight, oc, total)
    ii = jnp.arange(oc, dtype=jnp.int32)
    last_i = jnp.full((oc,), -1, jnp.int32).at[idx1].max(ii)
    written = (last_i >= 0).astype(jnp.int32)
    srcs = idx2[jnp.clip(last_i, 0, oc - 1)]

    xs = x[:, 0, :oc]
    x4 = x.reshape(B, oc, tpr, C)

    rb = _ROWS_PER_BLK
    grid = (B, oc // rb)
    blk = (1, rb, tpr, C)

    out4 = pl.pallas_call(
        functools.partial(_body, rb=rb, tpr=tpr, cc=C),
        grid=grid,
        in_specs=[
            pl.BlockSpec(memory_space=pltpu.SMEM),
            pl.BlockSpec(memory_space=pltpu.SMEM),
            pl.BlockSpec(memory_space=pltpu.SMEM),
            pl.BlockSpec(blk, lambda b, j: (b, j, 0, 0)),
        ],
        out_specs=pl.BlockSpec(blk, lambda b, j: (b, j, 0, 0)),
        out_shape=jax.ShapeDtypeStruct((B, oc, tpr, C), jnp.float32),
        compiler_params=pltpu.CompilerParams(
            dimension_semantics=("parallel", "arbitrary"),
        ),
    )(written, srcs, xs, x4)
    return out4.reshape(B, oc, ots)


# speedup vs baseline: 1.7479x; 1.0490x over previous
import functools

import jax
import jax.numpy as jnp
import numpy as np
from jax.experimental import pallas as pl
from jax.experimental.pallas import tpu as pltpu

_SCALE = 2
_ROWS_PER_BLK = 128


def _route_indices(weight, out_channels, total):
    t2 = np.float32(np.float64(total) * np.float64(total))
    p = weight[:out_channels].astype(jnp.float32) * t2
    t = jnp.trunc(p)
    oc = np.float32(out_channels)
    r = jnp.fmod(t, oc)
    r = jnp.where(r < 0, r + oc, r).astype(jnp.int32)
    return r[:, 0], r[:, 1]


def _body(w_ref, s_ref, xs_ref, x_ref, o_ref, *, rb, tpr, cc):
    b = pl.program_id(0)
    j = pl.program_id(1)
    for rr in range(rb):
        row = j * rb + rr

        @pl.when(w_ref[row] == 0)
        def _copy():
            o_ref[0, rr] = x_ref[0, rr]

        @pl.when(w_ref[row] != 0)
        def _fill():
            v = xs_ref[b, s_ref[row]]
            o_ref[0, rr] = jnp.full((tpr, cc), v, jnp.float32)


def kernel(x, weight):
    B, T, C = x.shape
    oc = C // _SCALE
    ots = T * _SCALE
    total = C * T - 1
    tpr = T // oc

    idx1, idx2 = _route_indices(weight, oc, total)
    ii = jnp.arange(oc, dtype=jnp.int32)
    last_i = jnp.full((oc,), -1, jnp.int32).at[idx1].max(ii)
    written = (last_i >= 0).astype(jnp.int32)
    srcs = idx2[jnp.clip(last_i, 0, oc - 1)]

    xs = x[:, 0, :oc]
    x4 = x.reshape(B, oc, tpr, C)

    rb = _ROWS_PER_BLK
    grid = (B, oc // rb)
    blk = (1, rb, tpr, C)

    out4 = pl.pallas_call(
        functools.partial(_body, rb=rb, tpr=tpr, cc=C),
        grid=grid,
        in_specs=[
            pl.BlockSpec(memory_space=pltpu.SMEM),
            pl.BlockSpec(memory_space=pltpu.SMEM),
            pl.BlockSpec(memory_space=pltpu.SMEM),
            pl.BlockSpec(blk, lambda b, j: (b, j, 0, 0)),
        ],
        out_specs=pl.BlockSpec(blk, lambda b, j: (b, j, 0, 0)),
        out_shape=jax.ShapeDtypeStruct((B, oc, tpr, C), jnp.float32),
        compiler_params=pltpu.CompilerParams(
            dimension_semantics=("parallel", "arbitrary"),
        ),
    )(written, srcs, xs, x4)
    return out4.reshape(B, oc, ots)


# speedup vs baseline: 1.7757x; 1.0159x over previous
import functools

import jax
import jax.numpy as jnp
import numpy as np
from jax.experimental import pallas as pl
from jax.experimental.pallas import tpu as pltpu

_SCALE = 2
_ROWS_PER_BLK = 256


def _route_indices(weight, out_channels, total):
    t2 = np.float32(np.float64(total) * np.float64(total))
    p = weight[:out_channels].astype(jnp.float32) * t2
    t = jnp.trunc(p)
    oc = np.float32(out_channels)
    r = jnp.fmod(t, oc)
    r = jnp.where(r < 0, r + oc, r).astype(jnp.int32)
    return r[:, 0], r[:, 1]


def _body(w_ref, s_ref, xs_ref, x_ref, o_ref, *, rb, tpr, cc):
    b = pl.program_id(0)
    j = pl.program_id(1)
    for rr in range(rb):
        row = j * rb + rr

        @pl.when(w_ref[row] == 0)
        def _copy():
            o_ref[0, rr] = x_ref[0, rr]

        @pl.when(w_ref[row] != 0)
        def _fill():
            v = xs_ref[b, s_ref[row]]
            o_ref[0, rr] = jnp.full((tpr, cc), v, jnp.float32)


def kernel(x, weight):
    B, T, C = x.shape
    oc = C // _SCALE
    ots = T * _SCALE
    total = C * T - 1
    tpr = T // oc

    idx1, idx2 = _route_indices(weight, oc, total)
    ii = jnp.arange(oc, dtype=jnp.int32)
    last_i = jnp.full((oc,), -1, jnp.int32).at[idx1].max(ii)
    written = (last_i >= 0).astype(jnp.int32)
    srcs = idx2[jnp.clip(last_i, 0, oc - 1)]

    xs = x[:, 0, :oc]
    x4 = x.reshape(B, oc, tpr, C)

    rb = _ROWS_PER_BLK
    grid = (B, oc // rb)
    blk = (1, rb, tpr, C)

    out4 = pl.pallas_call(
        functools.partial(_body, rb=rb, tpr=tpr, cc=C),
        grid=grid,
        in_specs=[
            pl.BlockSpec(memory_space=pltpu.SMEM),
            pl.BlockSpec(memory_space=pltpu.SMEM),
            pl.BlockSpec(memory_space=pltpu.SMEM),
            pl.BlockSpec(blk, lambda b, j: (b, j, 0, 0)),
        ],
        out_specs=pl.BlockSpec(blk, lambda b, j: (b, j, 0, 0)),
        out_shape=jax.ShapeDtypeStruct((B, oc, tpr, C), jnp.float32),
        compiler_params=pltpu.CompilerParams(
            dimension_semantics=("parallel", "arbitrary"),
        ),
    )(written, srcs, xs, x4)
    return out4.reshape(B, oc, ots)
